# baseline (speedup 1.0000x reference)
import functools

import jax
import jax.numpy as jnp
from jax import lax
from jax.experimental import pallas as pl
from jax.experimental.pallas import tpu as pltpu
from jax.experimental.pallas import tpu_sc as plsc

NUM_RNA = 6000
NUM_DIS = 3990
N_NODE = 10000
N_FEAT = 128
R = 64
N_PAIRS = 4096

AW = 128
N_ACC = 10112
NW = 32
MC = 128
MICROS = 84
E2P = MC * MICROS * NW
ROWS_PT = N_ACC // 16


def _dense1_body(x_ref, w1p_ref, ones_ref, haug_ref):
    xb = x_ref[...]
    haug_ref[...] = (jnp.dot(xb, w1p_ref[...],
                             preferred_element_type=jnp.float32)
                     + ones_ref[...])


def _dense1(x, W1p, ones_row):
    N = x.shape[0]
    blk = 2000
    grid = (N // blk,)
    return pl.pallas_call(
        _dense1_body,
        grid=grid,
        in_specs=[
            pl.BlockSpec((blk, N_FEAT), lambda i: (i, 0)),
            pl.BlockSpec((N_FEAT, 128), lambda i: (0, 0)),
            pl.BlockSpec((1, 128), lambda i: (0, 0)),
        ],
        out_specs=pl.BlockSpec((blk, 128), lambda i: (i, 0)),
        out_shape=jax.ShapeDtypeStruct((N, 128), jnp.float32),
    )(x, W1p, ones_row)


def _combine2_body(acc_ref, b1_ref, w2p_ref, ones_ref, haug_ref):
    a = acc_ref[0] + acc_ref[1]
    num = a[:, :R]
    den = a[:, R:R + 1]
    t = num / den
    haug = jnp.dot(t, w2p_ref[...], preferred_element_type=jnp.float32)
    haug = haug + jnp.dot(b1_ref[...], w2p_ref[...],
                          preferred_element_type=jnp.float32)
    haug_ref[...] = haug + ones_ref[...]


def _combine2(acc, b1v, W2p, ones_row):
    blk = 2000
    grid = (N_NODE // blk,)
    return pl.pallas_call(
        _combine2_body,
        grid=grid,
        in_specs=[
            pl.BlockSpec((2, blk, AW), lambda i: (0, i, 0)),
            pl.BlockSpec((1, R), lambda i: (0, 0)),
            pl.BlockSpec((R, 128), lambda i: (0, 0)),
            pl.BlockSpec((1, 128), lambda i: (0, 0)),
        ],
        out_specs=pl.BlockSpec((blk, 128), lambda i: (i, 0)),
        out_shape=jax.ShapeDtypeStruct((N_NODE, 128), jnp.float32),
    )(acc, b1v, W2p, ones_row)


def _mlp_body(rawl_ref, rawr_ref, b2cat_ref,
              w1, b1, w2, b2, w3, b3, w4, b4, w5, b5, w6, b6, o_ref):
    def lin(z, w, b):
        return jax.lax.dot_general(z, w[...], (((1,), (1,)), ((), ())),
                                   preferred_element_type=jnp.float32) + b[...]

    rawl = rawl_ref[...]
    rawr = rawr_ref[...]
    zl = rawl[:, :R] / rawl[:, R:R + 1]
    zr = rawr[:, :R] / rawr[:, R:R + 1]
    z = jnp.concatenate([zl, zr], axis=1) + b2cat_ref[...]
    z = jnp.maximum(lin(z, w1, b1), 0.0)
    z = jnp.maximum(lin(z, w2, b2), 0.0)
    z = jnp.maximum(lin(z, w3, b3), 0.0)
    z = jnp.maximum(lin(z, w4, b4), 0.0)
    z = jnp.maximum(lin(z, w5, b5), 0.0)
    o_ref[...] = jax.nn.sigmoid(lin(z, w6, b6))


def _mlp(rawl, rawr, b2cat, mWbs):
    P = rawl.shape[0]
    blk = 2048
    grid = (P // blk,)
    in_specs = [pl.BlockSpec((blk, 128), lambda i: (i, 0)),
                pl.BlockSpec((blk, 128), lambda i: (i, 0)),
                pl.BlockSpec((1, 128), lambda i: (0, 0))]
    args = [rawl, rawr, b2cat]
    for w, b in mWbs:
        in_specs.append(pl.BlockSpec(w.shape, lambda i: (0, 0)))
        in_specs.append(pl.BlockSpec(b.shape, lambda i: (0, 0)))
        args.extend([w, b])
    out = pl.pallas_call(
        _mlp_body,
        grid=grid,
        in_specs=in_specs,
        out_specs=pl.BlockSpec((blk, 128), lambda i: (i, 0)),
        out_shape=jax.ShapeDtypeStruct((P, 128), jnp.float32),
    )(*args)
    return out[:, :1]


_MESH = plsc.VectorSubcoreMesh(core_axis_name="c", subcore_axis_name="s")
_SC_PARAMS = pltpu.CompilerParams(needs_layout_passes=False)


@functools.partial(
    pl.kernel,
    out_type=jax.ShapeDtypeStruct((2, N_ACC, AW), jnp.float32),
    mesh=_MESH,
    compiler_params=_SC_PARAMS,
    scratch_types=[
        pltpu.VMEM((N_NODE,), jnp.float32),
        pltpu.VMEM((MICROS, MC), jnp.int32),
        pltpu.VMEM((MICROS, MC), jnp.int32),
        pltpu.VMEM((MC + 16,), jnp.float32),
        pltpu.VMEM((MC, AW), jnp.float32),
        pltpu.VMEM_SHARED((N_ACC, AW), jnp.float32),
        pltpu.SemaphoreType.DMA,
    ],
)
def _edge_kernel(h_hbm, ed_hbm, src3d_hbm, dst3d_hbm, out_hbm,
                 ed_t, src_i, dst_i, p_t, hb0, acc_sh, g0):
    cid = lax.axis_index("c")
    sid = lax.axis_index("s")
    wid = sid * 2 + cid

    def _zrow(j, _):
        zv = jnp.zeros((16,), jnp.float32)
        for cc in range(AW // 16):
            hb0[j, pl.ds(cc * 16, 16)] = zv
        return 0

    lax.fori_loop(0, MC, _zrow, 0)
    base_row = sid * ROWS_PT
    off = 0
    while off < ROWS_PT:
        n = min(MC, ROWS_PT - off)
        pltpu.sync_copy(hb0.at[pl.ds(0, n)],
                        acc_sh.at[pl.ds(base_row + off, n)])
        off += n
    plsc.subcore_barrier()

    pltpu.sync_copy(ed_hbm, ed_t)
    pltpu.sync_copy(src3d_hbm.at[wid], src_i)
    pltpu.sync_copy(dst3d_hbm.at[wid], dst_i)
    jv16 = jax.lax.iota(jnp.int32, 16)
    c65 = jnp.full((16,), 65, jnp.int32)

    def _micro(m, _):
        pltpu.async_copy(h_hbm.at[src_i.at[m]], hb0, g0).wait()

        for g in range(MC // 16):
            sl = pl.ds(g * 16, 16)
            dv = dst_i[m, sl]
            esv = plsc.load_gather(hb0, [jv16 + (g * 16), c65])
            e = esv + plsc.load_gather(ed_t, [dv])
            e = jnp.where(e >= 0, e, e * jnp.float32(0.2))
            p_t[sl] = jnp.exp(e)

        def _scale(j, _):
            pj = p_t[pl.ds(j, 16)][0]
            for cc in range(AW // 16):
                sl = pl.ds(cc * 16, 16)
                hb0[j, sl] = hb0[j, sl] * pj
            return 0

        lax.fori_loop(0, MC, _scale, 0)
        pltpu.sync_copy(hb0, acc_sh.at[dst_i.at[m]], add=True)
        return 0

    lax.fori_loop(0, MICROS, _micro, 0)

    plsc.subcore_barrier()
    pltpu.sync_copy(acc_sh.at[pl.ds(base_row, ROWS_PT)],
                    out_hbm.at[cid, pl.ds(base_row, ROWS_PT)])


@functools.partial(
    pl.kernel,
    out_type=(jax.ShapeDtypeStruct((N_PAIRS, 128), jnp.float32),
              jax.ShapeDtypeStruct((N_PAIRS, 128), jnp.float32)),
    mesh=_MESH,
    compiler_params=_SC_PARAMS,
    scratch_types=[
        pltpu.VMEM((128,), jnp.int32),
        pltpu.VMEM((128, AW), jnp.float32),
        pltpu.VMEM((128, AW), jnp.float32),
        pltpu.VMEM((128, 128), jnp.float32),
        pltpu.SemaphoreType.DMA,
        pltpu.SemaphoreType.DMA,
    ],
)
def _pair_kernel(accA_hbm, accB_hbm, ridx_hbm, didx_hbm,
                 rawl_hbm, rawr_hbm, idx_t, bufA, bufB, fbuf, semA, semB):
    cid = lax.axis_index("c")
    sid = lax.axis_index("s")
    wid = sid * 2 + cid
    pb = N_PAIRS // NW
    base = wid * pb

    for idx_hbm, out_hbm in ((ridx_hbm, rawl_hbm), (didx_hbm, rawr_hbm)):
        pltpu.sync_copy(idx_hbm.at[pl.ds(base, pb)], idx_t)
        pltpu.async_copy(accA_hbm.at[idx_t], bufA, semA).wait()
        pltpu.async_copy(accB_hbm.at[idx_t], bufB, semB).wait()

        def _row(j, _):
            for cc in range(AW // 16):
                sl = pl.ds(cc * 16, 16)
                fbuf[j, sl] = bufA[j, sl] + bufB[j, sl]
            return 0

        lax.fori_loop(0, pb, _row, 0)
        pltpu.sync_copy(fbuf, out_hbm.at[pl.ds(base, pb)])


def kernel(x, adj, coo_data, W1, a_s1, a_d1, b1, W2, a_s2, a_d2, b2,
           mW1, mb1, mW2, mb2, mW3, mb3, mW4, mb4, mW5, mb5, mW6, mb6):
    src, dst = adj[0], adj[1]
    loop = jnp.arange(N_NODE, dtype=src.dtype)
    npad = E2P - (src.shape[0] + N_NODE)
    pad_src = jnp.zeros((npad,), src.dtype)
    pad_dst = jnp.full((npad,), N_NODE, src.dtype) + (
        jnp.arange(npad, dtype=src.dtype) % 16)
    src2 = jnp.concatenate([src, loop, pad_src]).reshape(NW, MICROS, MC)
    dst2 = jnp.concatenate([dst, loop, pad_dst]).reshape(NW, MICROS, MC)

    ones_row = jnp.zeros((1, 128), jnp.float32).at[0, R].set(1.0)

    def _wpack(W, a_s, a_d):
        return jnp.concatenate(
            [W, jnp.zeros((W.shape[0], 1), jnp.float32),
             (W @ a_s)[:, None], (W @ a_d)[:, None],
             jnp.zeros((W.shape[0], 61), jnp.float32)], axis=1)

    W1p = _wpack(W1, a_s1, a_d1)
    W2p = _wpack(W2, a_s2, a_d2)

    haug1 = _dense1(x, W1p, ones_row)
    acc1 = _edge_kernel(haug1, haug1[:, 66], src2, dst2)

    haug2 = _combine2(acc1, b1.reshape(1, R), W2p, ones_row)
    acc2 = _edge_kernel(haug2, haug2[:, 66], src2, dst2)

    ridx = coo_data[:, 0]
    didx = coo_data[:, 1] + NUM_RNA
    rawl, rawr = _pair_kernel(acc2[0], acc2[1], ridx, didx)
    b2cat = jnp.concatenate([b2, b2]).reshape(1, 128)

    mWbs = []
    for w, b in ((mW1, mb1), (mW2, mb2), (mW3, mb3), (mW4, mb4),
                 (mW5, mb5), (mW6, mb6)):
        o, i = w.shape
        mWbs.append((jnp.pad(w, ((0, 128 - o), (0, 128 - i))),
                     jnp.pad(b.reshape(1, -1), ((0, 0), (0, 128 - o)))))
    return _mlp(rawl, rawr, b2cat, mWbs)


# speedup vs baseline: 1.2482x; 1.2482x over previous
import functools

import jax
import jax.numpy as jnp
from jax import lax
from jax.experimental import pallas as pl
from jax.experimental.pallas import tpu as pltpu
from jax.experimental.pallas import tpu_sc as plsc

NUM_RNA = 6000
NUM_DIS = 3990
N_NODE = 10000
N_FEAT = 128
R = 64
N_PAIRS = 4096

AW = 128
N_ACC = 10112
NW = 32
MC = 128
MICROS = 82
E2P = MC * MICROS * NW
ROWS_PT = N_ACC // 16


def _dense1_body(x_ref, w1p_ref, ones_ref, haug_ref):
    xb = x_ref[...]
    haug_ref[...] = (jnp.dot(xb, w1p_ref[...],
                             preferred_element_type=jnp.float32)
                     + ones_ref[...])


def _dense1(x, W1p, ones_row):
    N = x.shape[0]
    blk = 2000
    grid = (N // blk,)
    return pl.pallas_call(
        _dense1_body,
        grid=grid,
        in_specs=[
            pl.BlockSpec((blk, N_FEAT), lambda i: (i, 0)),
            pl.BlockSpec((N_FEAT, 128), lambda i: (0, 0)),
            pl.BlockSpec((1, 128), lambda i: (0, 0)),
        ],
        out_specs=pl.BlockSpec((blk, 128), lambda i: (i, 0)),
        out_shape=jax.ShapeDtypeStruct((N, 128), jnp.float32),
    )(x, W1p, ones_row)


def _combine2_body(acc_ref, b1_ref, w2p_ref, ones_ref, haug_ref):
    a = acc_ref[0] + acc_ref[1]
    num = a[:, :R]
    den = a[:, R:R + 1]
    t = num / den
    haug = jnp.dot(t, w2p_ref[...], preferred_element_type=jnp.float32)
    haug = haug + jnp.dot(b1_ref[...], w2p_ref[...],
                          preferred_element_type=jnp.float32)
    haug_ref[...] = haug + ones_ref[...]


def _combine2(acc, b1v, W2p, ones_row):
    blk = 2000
    grid = (N_NODE // blk,)
    return pl.pallas_call(
        _combine2_body,
        grid=grid,
        in_specs=[
            pl.BlockSpec((2, blk, AW), lambda i: (0, i, 0)),
            pl.BlockSpec((1, R), lambda i: (0, 0)),
            pl.BlockSpec((R, 128), lambda i: (0, 0)),
            pl.BlockSpec((1, 128), lambda i: (0, 0)),
        ],
        out_specs=pl.BlockSpec((blk, 128), lambda i: (i, 0)),
        out_shape=jax.ShapeDtypeStruct((N_NODE, 128), jnp.float32),
    )(acc, b1v, W2p, ones_row)


def _mlp_body(rawl_ref, rawr_ref, b2cat_ref,
              w1, b1, w2, b2, w3, b3, w4, b4, w5, b5, w6, b6, o_ref):
    def lin(z, w, b):
        return jax.lax.dot_general(z, w[...], (((1,), (1,)), ((), ())),
                                   preferred_element_type=jnp.float32) + b[...]

    rawl = rawl_ref[...]
    rawr = rawr_ref[...]
    zl = rawl[:, :R] / rawl[:, R:R + 1]
    zr = rawr[:, :R] / rawr[:, R:R + 1]
    z = jnp.concatenate([zl, zr], axis=1) + b2cat_ref[...]
    z = jnp.maximum(lin(z, w1, b1), 0.0)
    z = jnp.maximum(lin(z, w2, b2), 0.0)
    z = jnp.maximum(lin(z, w3, b3), 0.0)
    z = jnp.maximum(lin(z, w4, b4), 0.0)
    z = jnp.maximum(lin(z, w5, b5), 0.0)
    o_ref[...] = jax.nn.sigmoid(lin(z, w6, b6))


def _mlp(rawl, rawr, b2cat, mWbs):
    P = rawl.shape[0]
    blk = 2048
    grid = (P // blk,)
    in_specs = [pl.BlockSpec((blk, 128), lambda i: (i, 0)),
                pl.BlockSpec((blk, 128), lambda i: (i, 0)),
                pl.BlockSpec((1, 128), lambda i: (0, 0))]
    args = [rawl, rawr, b2cat]
    for w, b in mWbs:
        in_specs.append(pl.BlockSpec(w.shape, lambda i: (0, 0)))
        in_specs.append(pl.BlockSpec(b.shape, lambda i: (0, 0)))
        args.extend([w, b])
    out = pl.pallas_call(
        _mlp_body,
        grid=grid,
        in_specs=in_specs,
        out_specs=pl.BlockSpec((blk, 128), lambda i: (i, 0)),
        out_shape=jax.ShapeDtypeStruct((P, 128), jnp.float32),
    )(*args)
    return out[:, :1]


_MESH = plsc.VectorSubcoreMesh(core_axis_name="c", subcore_axis_name="s")
_SC_PARAMS = pltpu.CompilerParams(needs_layout_passes=False)


@functools.partial(
    pl.kernel,
    out_type=jax.ShapeDtypeStruct((2, N_ACC, AW), jnp.float32),
    mesh=_MESH,
    compiler_params=_SC_PARAMS,
    scratch_types=[
        pltpu.VMEM((N_NODE,), jnp.float32),
        pltpu.VMEM((N_NODE,), jnp.float32),
        pltpu.VMEM((MC,), jnp.int32),
        pltpu.VMEM((MC,), jnp.int32),
        pltpu.VMEM((MC + 16,), jnp.float32),
        pltpu.VMEM((MC, AW), jnp.float32),
        pltpu.VMEM_SHARED((N_ACC, AW), jnp.float32),
        pltpu.SemaphoreType.DMA,
    ],
)
def _edge_kernel(h_hbm, es_hbm, ed_hbm, src_hbm, dst_hbm, out_hbm,
                 es_t, ed_t, src_t, dst_t, p_t, hb0, acc_sh, g0):
    cid = lax.axis_index("c")
    sid = lax.axis_index("s")
    wid = sid * 2 + cid

    def _zrow(j, _):
        zv = jnp.zeros((16,), jnp.float32)
        for cc in range(AW // 16):
            hb0[j, pl.ds(cc * 16, 16)] = zv
        return 0

    lax.fori_loop(0, MC, _zrow, 0)
    base_row = sid * ROWS_PT
    off = 0
    while off < ROWS_PT:
        n = min(MC, ROWS_PT - off)
        pltpu.sync_copy(hb0.at[pl.ds(0, n)],
                        acc_sh.at[pl.ds(base_row + off, n)])
        off += n
    plsc.subcore_barrier()

    pltpu.sync_copy(es_hbm, es_t)
    pltpu.sync_copy(ed_hbm, ed_t)

    def _micro(m, _):
        ebase = (wid * MICROS + m) * MC
        pltpu.sync_copy(src_hbm.at[pl.ds(ebase, MC)], src_t)
        pltpu.sync_copy(dst_hbm.at[pl.ds(ebase, MC)], dst_t)
        pltpu.async_copy(h_hbm.at[src_t], hb0, g0).wait()

        for g in range(MC // 16):
            sl = pl.ds(g * 16, 16)
            sv = src_t[sl]
            dv = dst_t[sl]
            e = plsc.load_gather(es_t, [sv]) + plsc.load_gather(ed_t, [dv])
            e = jnp.where(e >= 0, e, e * jnp.float32(0.2))
            p_t[sl] = jnp.exp(e)

        def _scale(j, _):
            pj = p_t[pl.ds(j, 16)][0]
            for cc in range(AW // 16):
                sl = pl.ds(cc * 16, 16)
                hb0[j, sl] = hb0[j, sl] * pj
            return 0

        lax.fori_loop(0, MC, _scale, 0)
        pltpu.sync_copy(hb0, acc_sh.at[dst_t], add=True)
        return 0

    lax.fori_loop(0, MICROS, _micro, 0)

    plsc.subcore_barrier()
    pltpu.sync_copy(acc_sh.at[pl.ds(base_row, ROWS_PT)],
                    out_hbm.at[cid, pl.ds(base_row, ROWS_PT)])


@functools.partial(
    pl.kernel,
    out_type=(jax.ShapeDtypeStruct((N_PAIRS, 128), jnp.float32),
              jax.ShapeDtypeStruct((N_PAIRS, 128), jnp.float32)),
    mesh=_MESH,
    compiler_params=_SC_PARAMS,
    scratch_types=[
        pltpu.VMEM((128,), jnp.int32),
        pltpu.VMEM((128, AW), jnp.float32),
        pltpu.VMEM((128, AW), jnp.float32),
        pltpu.VMEM((128, 128), jnp.float32),
        pltpu.SemaphoreType.DMA,
        pltpu.SemaphoreType.DMA,
    ],
)
def _pair_kernel(accA_hbm, accB_hbm, ridx_hbm, didx_hbm,
                 rawl_hbm, rawr_hbm, idx_t, bufA, bufB, fbuf, semA, semB):
    cid = lax.axis_index("c")
    sid = lax.axis_index("s")
    wid = sid * 2 + cid
    pb = N_PAIRS // NW
    base = wid * pb

    for idx_hbm, out_hbm in ((ridx_hbm, rawl_hbm), (didx_hbm, rawr_hbm)):
        pltpu.sync_copy(idx_hbm.at[pl.ds(base, pb)], idx_t)
        pltpu.async_copy(accA_hbm.at[idx_t], bufA, semA).wait()
        pltpu.async_copy(accB_hbm.at[idx_t], bufB, semB).wait()

        def _row(j, _):
            for cc in range(AW // 16):
                sl = pl.ds(cc * 16, 16)
                fbuf[j, sl] = bufA[j, sl] + bufB[j, sl]
            return 0

        lax.fori_loop(0, pb, _row, 0)
        pltpu.sync_copy(fbuf, out_hbm.at[pl.ds(base, pb)])


def kernel(x, adj, coo_data, W1, a_s1, a_d1, b1, W2, a_s2, a_d2, b2,
           mW1, mb1, mW2, mb2, mW3, mb3, mW4, mb4, mW5, mb5, mW6, mb6):
    src, dst = adj[0], adj[1]
    loop = jnp.arange(N_NODE, dtype=src.dtype)
    npad = E2P - (src.shape[0] + N_NODE)
    pad_src = jnp.zeros((npad,), src.dtype)
    pad_dst = jnp.full((npad,), N_NODE, src.dtype) + (
        jnp.arange(npad, dtype=src.dtype) % 16)
    src2 = jnp.concatenate([src, loop, pad_src])
    dst2 = jnp.concatenate([dst, loop, pad_dst])

    ones_row = jnp.zeros((1, 128), jnp.float32).at[0, R].set(1.0)

    def _wpack(W, a_s, a_d):
        return jnp.concatenate(
            [W, jnp.zeros((W.shape[0], 1), jnp.float32),
             (W @ a_s)[:, None], (W @ a_d)[:, None],
             jnp.zeros((W.shape[0], 61), jnp.float32)], axis=1)

    W1p = _wpack(W1, a_s1, a_d1)
    W2p = _wpack(W2, a_s2, a_d2)

    haug1 = _dense1(x, W1p, ones_row)
    acc1 = _edge_kernel(haug1, haug1[:, 65], haug1[:, 66], src2, dst2)

    haug2 = _combine2(acc1, b1.reshape(1, R), W2p, ones_row)
    acc2 = _edge_kernel(haug2, haug2[:, 65], haug2[:, 66], src2, dst2)

    ridx = coo_data[:, 0]
    didx = coo_data[:, 1] + NUM_RNA
    rawl, rawr = _pair_kernel(acc2[0], acc2[1], ridx, didx)
    b2cat = jnp.concatenate([b2, b2]).reshape(1, 128)

    mWbs = []
    for w, b in ((mW1, mb1), (mW2, mb2), (mW3, mb3), (mW4, mb4),
                 (mW5, mb5), (mW6, mb6)):
        o, i = w.shape
        mWbs.append((jnp.pad(w, ((0, 128 - o), (0, 128 - i))),
                     jnp.pad(b.reshape(1, -1), ((0, 0), (0, 128 - o)))))
    return _mlp(rawl, rawr, b2cat, mWbs)


# speedup vs baseline: 1.3761x; 1.1025x over previous
import functools

import jax
import jax.numpy as jnp
from jax import lax
from jax.experimental import pallas as pl
from jax.experimental.pallas import tpu as pltpu
from jax.experimental.pallas import tpu_sc as plsc

NUM_RNA = 6000
NUM_DIS = 3990
N_NODE = 10000
N_FEAT = 128
R = 64
N_PAIRS = 4096

AW = 128
N_ACC = 10112
NW = 32
MC = 128
MICROS = 82
E2P = MC * MICROS * NW
ROWS_PT = N_ACC // 16


def _dense1_body(x_ref, w1p_ref, ones_ref, haug_ref):
    xb = x_ref[...]
    haug_ref[...] = (jnp.dot(xb, w1p_ref[...],
                             preferred_element_type=jnp.float32)
                     + ones_ref[...])


def _dense1(x, W1p, ones_row):
    N = x.shape[0]
    blk = 2000
    grid = (N // blk,)
    return pl.pallas_call(
        _dense1_body,
        grid=grid,
        in_specs=[
            pl.BlockSpec((blk, N_FEAT), lambda i: (i, 0)),
            pl.BlockSpec((N_FEAT, 128), lambda i: (0, 0)),
            pl.BlockSpec((1, 128), lambda i: (0, 0)),
        ],
        out_specs=pl.BlockSpec((blk, 128), lambda i: (i, 0)),
        out_shape=jax.ShapeDtypeStruct((N, 128), jnp.float32),
    )(x, W1p, ones_row)


def _combine2_body(acc_ref, b1_ref, w2p_ref, ones_ref, haug_ref):
    a = acc_ref[0] + acc_ref[1]
    num = a[:, :R]
    den = a[:, R:R + 1]
    t = num / den
    haug = jnp.dot(t, w2p_ref[...], preferred_element_type=jnp.float32)
    haug = haug + jnp.dot(b1_ref[...], w2p_ref[...],
                          preferred_element_type=jnp.float32)
    haug_ref[...] = haug + ones_ref[...]


def _combine2(acc, b1v, W2p, ones_row):
    blk = 2000
    grid = (N_NODE // blk,)
    return pl.pallas_call(
        _combine2_body,
        grid=grid,
        in_specs=[
            pl.BlockSpec((2, blk, AW), lambda i: (0, i, 0)),
            pl.BlockSpec((1, R), lambda i: (0, 0)),
            pl.BlockSpec((R, 128), lambda i: (0, 0)),
            pl.BlockSpec((1, 128), lambda i: (0, 0)),
        ],
        out_specs=pl.BlockSpec((blk, 128), lambda i: (i, 0)),
        out_shape=jax.ShapeDtypeStruct((N_NODE, 128), jnp.float32),
    )(acc, b1v, W2p, ones_row)


def _mlp_body(rawl_ref, rawr_ref, b2cat_ref,
              w1, b1, w2, b2, w3, b3, w4, b4, w5, b5, w6, b6, o_ref):
    def lin(z, w, b):
        return jax.lax.dot_general(z, w[...], (((1,), (1,)), ((), ())),
                                   preferred_element_type=jnp.float32) + b[...]

    rawl = rawl_ref[...]
    rawr = rawr_ref[...]
    zl = rawl[:, :R] / rawl[:, R:R + 1]
    zr = rawr[:, :R] / rawr[:, R:R + 1]
    z = jnp.concatenate([zl, zr], axis=1) + b2cat_ref[...]
    z = jnp.maximum(lin(z, w1, b1), 0.0)
    z = jnp.maximum(lin(z, w2, b2), 0.0)
    z = jnp.maximum(lin(z, w3, b3), 0.0)
    z = jnp.maximum(lin(z, w4, b4), 0.0)
    z = jnp.maximum(lin(z, w5, b5), 0.0)
    o_ref[...] = jax.nn.sigmoid(lin(z, w6, b6))


def _mlp(rawl, rawr, b2cat, mWbs):
    P = rawl.shape[0]
    blk = 2048
    grid = (P // blk,)
    in_specs = [pl.BlockSpec((blk, 128), lambda i: (i, 0)),
                pl.BlockSpec((blk, 128), lambda i: (i, 0)),
                pl.BlockSpec((1, 128), lambda i: (0, 0))]
    args = [rawl, rawr, b2cat]
    for w, b in mWbs:
        in_specs.append(pl.BlockSpec(w.shape, lambda i: (0, 0)))
        in_specs.append(pl.BlockSpec(b.shape, lambda i: (0, 0)))
        args.extend([w, b])
    out = pl.pallas_call(
        _mlp_body,
        grid=grid,
        in_specs=in_specs,
        out_specs=pl.BlockSpec((blk, 128), lambda i: (i, 0)),
        out_shape=jax.ShapeDtypeStruct((P, 128), jnp.float32),
    )(*args)
    return out[:, :1]


_MESH = plsc.VectorSubcoreMesh(core_axis_name="c", subcore_axis_name="s")
_SC_PARAMS = pltpu.CompilerParams(needs_layout_passes=False)


@functools.partial(
    pl.kernel,
    out_type=jax.ShapeDtypeStruct((2, N_ACC, AW), jnp.float32),
    mesh=_MESH,
    compiler_params=_SC_PARAMS,
    scratch_types=[
        pltpu.VMEM((N_NODE,), jnp.float32),
        pltpu.VMEM((N_NODE,), jnp.float32),
        pltpu.VMEM((MC,), jnp.int32),
        pltpu.VMEM((MC,), jnp.int32),
        pltpu.VMEM((MC + 16,), jnp.float32),
        pltpu.VMEM((MC, AW), jnp.float32),
        pltpu.VMEM_SHARED((N_ACC, AW), jnp.float32),
        pltpu.SemaphoreType.DMA,
    ],
)
def _edge_kernel(h_hbm, es_hbm, ed_hbm, src_hbm, dst_hbm, out_hbm,
                 es_t, ed_t, src_t, dst_t, p_t, hb0, acc_sh, g0):
    cid = lax.axis_index("c")
    sid = lax.axis_index("s")
    wid = sid * 2 + cid

    def _zrow(j, _):
        zv = jnp.zeros((16,), jnp.float32)
        for cc in range(AW // 16):
            hb0[j, pl.ds(cc * 16, 16)] = zv
        return 0

    lax.fori_loop(0, MC, _zrow, 0)
    base_row = sid * ROWS_PT
    off = 0
    while off < ROWS_PT:
        n = min(MC, ROWS_PT - off)
        pltpu.sync_copy(hb0.at[pl.ds(0, n)],
                        acc_sh.at[pl.ds(base_row + off, n)])
        off += n
    plsc.subcore_barrier()

    pltpu.sync_copy(es_hbm, es_t)
    pltpu.sync_copy(ed_hbm, ed_t)

    def _micro(m, _):
        ebase = (wid * MICROS + m) * MC
        pltpu.sync_copy(src_hbm.at[pl.ds(ebase, MC)], src_t)
        pltpu.sync_copy(dst_hbm.at[pl.ds(ebase, MC)], dst_t)
        gather = pltpu.async_copy(h_hbm.at[src_t], hb0, g0)

        for g in range(MC // 16):
            sl = pl.ds(g * 16, 16)
            sv = src_t[sl]
            dv = dst_t[sl]
            e = plsc.load_gather(es_t, [sv]) + plsc.load_gather(ed_t, [dv])
            e = jnp.where(e >= 0, e, e * jnp.float32(0.2))
            p_t[sl] = jnp.exp(e)

        gather.wait()

        @plsc.parallel_loop(0, MC, unroll=2)
        def _scale(j):
            pj = p_t[pl.ds(j, 16)][0]
            for cc in range(AW // 16):
                sl = pl.ds(cc * 16, 16)
                hb0[j, sl] = hb0[j, sl] * pj
        pltpu.sync_copy(hb0, acc_sh.at[dst_t], add=True)
        return 0

    lax.fori_loop(0, MICROS, _micro, 0)

    plsc.subcore_barrier()
    pltpu.sync_copy(acc_sh.at[pl.ds(base_row, ROWS_PT)],
                    out_hbm.at[cid, pl.ds(base_row, ROWS_PT)])


@functools.partial(
    pl.kernel,
    out_type=(jax.ShapeDtypeStruct((N_PAIRS, 128), jnp.float32),
              jax.ShapeDtypeStruct((N_PAIRS, 128), jnp.float32)),
    mesh=_MESH,
    compiler_params=_SC_PARAMS,
    scratch_types=[
        pltpu.VMEM((128,), jnp.int32),
        pltpu.VMEM((128, AW), jnp.float32),
        pltpu.VMEM((128, AW), jnp.float32),
        pltpu.VMEM((128, 128), jnp.float32),
        pltpu.SemaphoreType.DMA,
        pltpu.SemaphoreType.DMA,
    ],
)
def _pair_kernel(accA_hbm, accB_hbm, ridx_hbm, didx_hbm,
                 rawl_hbm, rawr_hbm, idx_t, bufA, bufB, fbuf, semA, semB):
    cid = lax.axis_index("c")
    sid = lax.axis_index("s")
    wid = sid * 2 + cid
    pb = N_PAIRS // NW
    base = wid * pb

    for idx_hbm, out_hbm in ((ridx_hbm, rawl_hbm), (didx_hbm, rawr_hbm)):
        pltpu.sync_copy(idx_hbm.at[pl.ds(base, pb)], idx_t)
        pltpu.async_copy(accA_hbm.at[idx_t], bufA, semA).wait()
        pltpu.async_copy(accB_hbm.at[idx_t], bufB, semB).wait()

        def _row(j, _):
            for cc in range(AW // 16):
                sl = pl.ds(cc * 16, 16)
                fbuf[j, sl] = bufA[j, sl] + bufB[j, sl]
            return 0

        lax.fori_loop(0, pb, _row, 0)
        pltpu.sync_copy(fbuf, out_hbm.at[pl.ds(base, pb)])


def kernel(x, adj, coo_data, W1, a_s1, a_d1, b1, W2, a_s2, a_d2, b2,
           mW1, mb1, mW2, mb2, mW3, mb3, mW4, mb4, mW5, mb5, mW6, mb6):
    src, dst = adj[0], adj[1]
    loop = jnp.arange(N_NODE, dtype=src.dtype)
    npad = E2P - (src.shape[0] + N_NODE)
    pad_src = jnp.zeros((npad,), src.dtype)
    pad_dst = jnp.full((npad,), N_NODE, src.dtype) + (
        jnp.arange(npad, dtype=src.dtype) % 16)
    src2 = jnp.concatenate([src, loop, pad_src])
    dst2 = jnp.concatenate([dst, loop, pad_dst])

    ones_row = jnp.zeros((1, 128), jnp.float32).at[0, R].set(1.0)

    def _wpack(W, a_s, a_d):
        return jnp.concatenate(
            [W, jnp.zeros((W.shape[0], 1), jnp.float32),
             (W @ a_s)[:, None], (W @ a_d)[:, None],
             jnp.zeros((W.shape[0], 61), jnp.float32)], axis=1)

    W1p = _wpack(W1, a_s1, a_d1)
    W2p = _wpack(W2, a_s2, a_d2)

    haug1 = _dense1(x, W1p, ones_row)
    acc1 = _edge_kernel(haug1, haug1[:, 65], haug1[:, 66], src2, dst2)

    haug2 = _combine2(acc1, b1.reshape(1, R), W2p, ones_row)
    acc2 = _edge_kernel(haug2, haug2[:, 65], haug2[:, 66], src2, dst2)

    ridx = coo_data[:, 0]
    didx = coo_data[:, 1] + NUM_RNA
    rawl, rawr = _pair_kernel(acc2[0], acc2[1], ridx, didx)
    b2cat = jnp.concatenate([b2, b2]).reshape(1, 128)

    mWbs = []
    for w, b in ((mW1, mb1), (mW2, mb2), (mW3, mb3), (mW4, mb4),
                 (mW5, mb5), (mW6, mb6)):
        o, i = w.shape
        mWbs.append((jnp.pad(w, ((0, 128 - o), (0, 128 - i))),
                     jnp.pad(b.reshape(1, -1), ((0, 0), (0, 128 - o)))))
    return _mlp(rawl, rawr, b2cat, mWbs)
